# baseline (speedup 1.0000x reference)
import jax
import jax.numpy as jnp
from jax import lax
from jax.experimental import pallas as pl
from jax.experimental.pallas import tpu as pltpu
from jax.experimental.pallas import tpu_sc as plsc

L = 16
C = 32768
NV = C // L
G = 16
ROWS = 1024
NW = 32
KTOP = 8
NEG = float("-inf")
NCH = 4
CHUNK = ROWS // NCH
RPWC = CHUNK // NW
R = 8
NB = ROWS // R
BB = 4


def _tree_max(vs):
    vs = list(vs)
    while len(vs) > 1:
        nxt = [jnp.maximum(vs[i], vs[i + 1]) for i in range(0, len(vs) - 1, 2)]
        if len(vs) % 2:
            nxt.append(vs[-1])
        vs = nxt
    return vs[0]


def _sc_thr_body(chunk_base):
    def body(in_hbm, thr_hbm, bufA, bufB, thr_v, lsemA, lsemB):
        wid = lax.axis_index("s") * 2 + lax.axis_index("c")
        base = chunk_base + wid * RPWC
        last = base + RPWC - 1

        pltpu.async_copy(in_hbm.at[base], bufA, lsemA)
        pltpu.async_copy(in_hbm.at[base + 1], bufB, lsemB)

        def process(i, buf, lsem):
            row = base + i
            pltpu.make_async_copy(in_hbm.at[row], buf, lsem).wait()

            init = tuple(jnp.full((L,), NEG) for _ in range(2 * G))

            def p1(j, st):
                cs = list(st[:G])
                ds = list(st[G:])
                for g in range(G):
                    v = buf[pl.ds(j * G * L + g * L, L)]
                    lo = jnp.minimum(cs[g], v)
                    cs[g] = jnp.maximum(cs[g], v)
                    ds[g] = jnp.maximum(ds[g], lo)
                return tuple(cs) + tuple(ds)

            pool = lax.fori_loop(0, NV // G, p1, init)

            t = jnp.float32(float("inf"))
            for _ in range(KTOP):
                masked = [jnp.where(p < t, p, NEG) for p in pool]
                t = jnp.max(_tree_max(masked))
            thr_v[pl.ds(i * L, L)] = jnp.full((L,), t)

            nxt = jnp.minimum(row + 2, last)
            pltpu.async_copy(in_hbm.at[nxt], buf, lsem)

        def pair(i, carry):
            process(2 * i, bufA, lsemA)
            process(2 * i + 1, bufB, lsemB)
            return carry

        lax.fori_loop(0, RPWC // 2, pair, jnp.int32(0))

        pltpu.make_async_copy(in_hbm.at[last], bufA, lsemA).wait()
        pltpu.make_async_copy(in_hbm.at[last], bufB, lsemB).wait()
        pltpu.sync_copy(thr_v, thr_hbm.at[pl.ds(wid * RPWC * L, RPWC * L)])

    return body


def _sc_thresholds(flat, chunk_base):
    mesh = plsc.VectorSubcoreMesh(core_axis_name="c", subcore_axis_name="s")
    return pl.kernel(
        _sc_thr_body(chunk_base),
        out_type=jax.ShapeDtypeStruct((CHUNK * L,), jnp.float32),
        mesh=mesh,
        compiler_params=pltpu.CompilerParams(needs_layout_passes=False),
        scratch_types=[
            pltpu.VMEM((C,), jnp.float32),
            pltpu.VMEM((C,), jnp.float32),
            pltpu.VMEM((RPWC * L,), jnp.float32),
            pltpu.SemaphoreType.DMA,
            pltpu.SemaphoreType.DMA,
        ],
    )(flat)


def _tc_mask_body(*refs):
    x_ref, thr_ref, kv_ref = refs[0], refs[1], refs[2]
    o_ref = refs[-1]
    kvs = kv_ref[0, 0, 0]
    for b in range(BB):
        x = x_ref[b]
        tv = thr_ref[b]
        m = x >= tv
        cnt = jnp.sum(m.astype(jnp.int32), axis=1, keepdims=True)
        o_ref[b] = jnp.where(m, x, 0.0) + kvs
        for r in range(R):
            @pl.when(cnt[r, 0] != KTOP)
            def _repair(b=b, r=r, x=x):
                xr = x[r:r + 1, :]

                def cond(st):
                    return st[1] < KTOP

                def wbody(st):
                    tc, cgt, _tp, _cp = st
                    mv = jnp.max(jnp.where(xr < tc, xr, NEG))
                    ce = jnp.sum((xr == tc).astype(jnp.int32))
                    return (mv, cgt + ce, tc, cgt)

                tinf = jnp.float32(float("inf"))
                st = lax.while_loop(
                    cond, wbody, (tinf, jnp.int32(0), tinf, jnp.int32(0)))
                t8x = st[2]
                need = KTOP - st[3]
                eq = xr == t8x
                flat_i = lax.broadcasted_iota(jnp.int32, (1, C), 1)
                big = jnp.int32(C)

                def mstep(j, cutoff):
                    m2 = jnp.logical_and(eq, flat_i > cutoff)
                    nxt = jnp.min(jnp.where(m2, flat_i, big))
                    return jnp.where(j < need, nxt, cutoff)

                cutoff = lax.fori_loop(0, KTOP, mstep, jnp.int32(-1))
                keep = jnp.logical_or(
                    xr > t8x, jnp.logical_and(eq, flat_i <= cutoff))
                o_ref[b, pl.ds(r, 1), :] = jnp.where(keep, xr, 0.0) + kvs


def _tc_mask(x, thr3d, kv3d, chunk_base, prev):
    bb = (chunk_base // R) // BB
    in_specs = [
        pl.BlockSpec((BB, R, C), lambda i: (bb + i, 0, 0)),
        pl.BlockSpec((BB, R, 1), lambda i: (i, 0, 0)),
        pl.BlockSpec((1, 1, L), lambda i: (0, 0, 0)),
    ]
    args = [x, thr3d, kv3d]
    aliases = {}
    if prev is not None:
        in_specs.append(pl.BlockSpec(memory_space=pl.ANY))
        args.append(prev)
        aliases = {3: 0}
    return pl.pallas_call(
        _tc_mask_body,
        grid=(CHUNK // (BB * R),),
        in_specs=in_specs,
        out_specs=pl.BlockSpec((BB, R, C), lambda i: (bb + i, 0, 0)),
        out_shape=jax.ShapeDtypeStruct((NB, R, C), jnp.float32),
        input_output_aliases=aliases,
        compiler_params=pltpu.CompilerParams(
            dimension_semantics=("arbitrary",)),
    )(*args)


def kernel(t, K):
    flat = t.reshape(ROWS, C)
    kv3d = jnp.full((1, 1, L), jnp.asarray(K, jnp.float32) - float(KTOP),
                    dtype=jnp.float32)

    out = None
    for c in range(NCH):
        thr = _sc_thresholds(flat, c * CHUNK)
        thr3d = thr.reshape(CHUNK, L)[:, :1].reshape(CHUNK // R, R, 1)
        out = _tc_mask(t, thr3d, kv3d, c * CHUNK, out)
    return out


# speedup vs baseline: 2.7357x; 2.7357x over previous
import jax
import jax.numpy as jnp
from jax import lax
from jax.experimental import pallas as pl
from jax.experimental.pallas import tpu as pltpu
from jax.experimental.pallas import tpu_sc as plsc

L = 16
C = 32768
NV = C // L
G = 16
ROWS = 1024
NW = 32
KTOP = 8
NEG = float("-inf")
NCH = 4
CHUNK = ROWS // NCH
RPWC = CHUNK // NW
R = 8
NB = ROWS // R
BB = 4


def _tree_max(vs):
    vs = list(vs)
    while len(vs) > 1:
        nxt = [jnp.maximum(vs[i], vs[i + 1]) for i in range(0, len(vs) - 1, 2)]
        if len(vs) % 2:
            nxt.append(vs[-1])
        vs = nxt
    return vs[0]


def _sc_thr_body(chunk_base):
    def body(in_hbm, thr_hbm, bufA, bufB, thr_v, lsemA, lsemB):
        wid = lax.axis_index("s") * 2 + lax.axis_index("c")
        base = chunk_base + wid * RPWC
        last = base + RPWC - 1

        pltpu.async_copy(in_hbm.at[base], bufA, lsemA)
        pltpu.async_copy(in_hbm.at[base + 1], bufB, lsemB)

        def process(i, buf, lsem):
            row = base + i
            pltpu.make_async_copy(in_hbm.at[row], buf, lsem).wait()

            init = tuple(jnp.full((L,), NEG) for _ in range(2 * G))

            def p1(j, st):
                cs = list(st[:G])
                ds = list(st[G:])
                for g in range(G):
                    v = buf[pl.ds(j * G * L + g * L, L)]
                    lo = jnp.minimum(cs[g], v)
                    cs[g] = jnp.maximum(cs[g], v)
                    ds[g] = jnp.maximum(ds[g], lo)
                return tuple(cs) + tuple(ds)

            pool = lax.fori_loop(0, NV // G, p1, init)

            t = jnp.float32(float("inf"))
            for _ in range(KTOP):
                masked = [jnp.where(p < t, p, NEG) for p in pool]
                t = jnp.max(_tree_max(masked))
            thr_v[pl.ds(i * L, L)] = jnp.full((L,), t)

            nxt = jnp.minimum(row + 2, last)
            pltpu.async_copy(in_hbm.at[nxt], buf, lsem)

        def pair(i, carry):
            process(2 * i, bufA, lsemA)
            process(2 * i + 1, bufB, lsemB)
            return carry

        lax.fori_loop(0, RPWC // 2, pair, jnp.int32(0))

        pltpu.make_async_copy(in_hbm.at[last], bufA, lsemA).wait()
        pltpu.make_async_copy(in_hbm.at[last], bufB, lsemB).wait()
        pltpu.sync_copy(thr_v, thr_hbm.at[pl.ds(wid * RPWC * L, RPWC * L)])

    return body


def _sc_thresholds(flat, chunk_base):
    mesh = plsc.VectorSubcoreMesh(core_axis_name="c", subcore_axis_name="s")
    return pl.kernel(
        _sc_thr_body(chunk_base),
        out_type=jax.ShapeDtypeStruct((CHUNK * L,), jnp.float32),
        mesh=mesh,
        compiler_params=pltpu.CompilerParams(needs_layout_passes=False),
        scratch_types=[
            pltpu.VMEM((C,), jnp.float32),
            pltpu.VMEM((C,), jnp.float32),
            pltpu.VMEM((RPWC * L,), jnp.float32),
            pltpu.SemaphoreType.DMA,
            pltpu.SemaphoreType.DMA,
        ],
    )(flat)


def _tc_mask_body(*refs):
    x_ref, thr_ref, kv_ref = refs[0], refs[1], refs[2]
    o_ref = refs[-1]
    kvs = kv_ref[0, 0, 0]
    cnts = []
    for b in range(BB):
        x = x_ref[b]
        tv = thr_ref[b]
        m = x >= tv
        cnt = jnp.sum(m.astype(jnp.int32), axis=1, keepdims=True)
        cnts.append(cnt)
        o_ref[b] = jnp.where(m, x, 0.0) + kvs
    nbad = sum(jnp.sum((c != KTOP).astype(jnp.int32)) for c in cnts)

    @pl.when(nbad > 0)
    def _repair_block():
        for b in range(BB):
            for r in range(R):
                @pl.when(cnts[b][r, 0] != KTOP)
                def _repair(b=b, r=r):
                    xr = x_ref[b, r:r + 1, :]

                    def cond(st):
                        return st[1] < KTOP

                    def wbody(st):
                        tc, cgt, _tp, _cp = st
                        mv = jnp.max(jnp.where(xr < tc, xr, NEG))
                        ce = jnp.sum((xr == tc).astype(jnp.int32))
                        return (mv, cgt + ce, tc, cgt)

                    tinf = jnp.float32(float("inf"))
                    st = lax.while_loop(
                        cond, wbody,
                        (tinf, jnp.int32(0), tinf, jnp.int32(0)))
                    t8x = st[2]
                    need = KTOP - st[3]
                    eq = xr == t8x
                    flat_i = lax.broadcasted_iota(jnp.int32, (1, C), 1)
                    big = jnp.int32(C)

                    def mstep(j, cutoff):
                        m2 = jnp.logical_and(eq, flat_i > cutoff)
                        nxt = jnp.min(jnp.where(m2, flat_i, big))
                        return jnp.where(j < need, nxt, cutoff)

                    cutoff = lax.fori_loop(0, KTOP, mstep, jnp.int32(-1))
                    keep = jnp.logical_or(
                        xr > t8x, jnp.logical_and(eq, flat_i <= cutoff))
                    o_ref[b, pl.ds(r, 1), :] = (
                        jnp.where(keep, xr, 0.0) + kvs)


def _tc_mask(x, thr3d, kv3d, chunk_base, prev):
    bb = (chunk_base // R) // BB
    in_specs = [
        pl.BlockSpec((BB, R, C), lambda i: (bb + i, 0, 0)),
        pl.BlockSpec((BB, R, 1), lambda i: (i, 0, 0)),
        pl.BlockSpec((1, 1, L), lambda i: (0, 0, 0)),
    ]
    args = [x, thr3d, kv3d]
    aliases = {}
    if prev is not None:
        in_specs.append(pl.BlockSpec(memory_space=pl.ANY))
        args.append(prev)
        aliases = {3: 0}
    return pl.pallas_call(
        _tc_mask_body,
        grid=(CHUNK // (BB * R),),
        in_specs=in_specs,
        out_specs=pl.BlockSpec((BB, R, C), lambda i: (bb + i, 0, 0)),
        out_shape=jax.ShapeDtypeStruct((NB, R, C), jnp.float32),
        input_output_aliases=aliases,
        compiler_params=pltpu.CompilerParams(
            dimension_semantics=("arbitrary",)),
    )(*args)


def kernel(t, K):
    flat = t.reshape(ROWS, C)
    kv3d = jnp.full((1, 1, L), jnp.asarray(K, jnp.float32) - float(KTOP),
                    dtype=jnp.float32)

    out = None
    for c in range(NCH):
        thr = _sc_thresholds(flat, c * CHUNK)
        thr3d = thr.reshape(CHUNK, L)[:, :1].reshape(CHUNK // R, R, 1)
        out = _tc_mask(t, thr3d, kv3d, c * CHUNK, out)
    return out


# speedup vs baseline: 5.4275x; 1.9839x over previous
import jax
import jax.numpy as jnp
from jax import lax
from jax.experimental import pallas as pl
from jax.experimental.pallas import tpu as pltpu
from jax.experimental.pallas import tpu_sc as plsc

L = 16
C = 32768
NV = C // L
G = 16
ROWS = 1024
NW = 32
RPW = ROWS // NW
KTOP = 8
NEG = float("-inf")


def _tree_max(vs):
    vs = list(vs)
    while len(vs) > 1:
        nxt = [jnp.maximum(vs[i], vs[i + 1]) for i in range(0, len(vs) - 1, 2)]
        if len(vs) % 2:
            nxt.append(vs[-1])
        vs = nxt
    return vs[0]


def _sc_body(in_hbm, k_hbm, out_hbm, bufA, bufB, kv_v, lsemA, lsemB, ssem):
    wid = lax.axis_index("s") * 2 + lax.axis_index("c")
    base = wid * RPW
    last = base + RPW - 1
    pltpu.sync_copy(k_hbm, kv_v)
    kv = kv_v[...]

    pltpu.async_copy(in_hbm.at[base], bufA, lsemA)
    pltpu.async_copy(in_hbm.at[base + 1], bufB, lsemB)

    def process(row, buf, lsem, other, olsem):
        pltpu.make_async_copy(in_hbm.at[row], buf, lsem).wait()

        init = tuple(jnp.full((L,), NEG) for _ in range(2 * G))

        def p1(j, st):
            cs = list(st[:G])
            ds = list(st[G:])
            for g in range(G):
                v = buf[pl.ds(j * G * L + g * L, L)]
                lo = jnp.minimum(cs[g], v)
                cs[g] = jnp.maximum(cs[g], v)
                ds[g] = jnp.maximum(ds[g], lo)
            return tuple(cs) + tuple(ds)

        pool = lax.fori_loop(0, NV // G, p1, init)

        t = jnp.float32(float("inf"))
        for _ in range(KTOP):
            masked = [jnp.where(p < t, p, NEG) for p in pool]
            t = jnp.max(_tree_max(masked))
        t8v = jnp.full((L,), t)

        @pl.when(row > base)
        def _pump():
            pltpu.make_async_copy(other, out_hbm.at[row], ssem).wait()
            nxt = jnp.minimum(row + 1, last)
            pltpu.async_copy(in_hbm.at[nxt], other, olsem)

        U = 8

        def p2(j, cnt):
            for u in range(U):
                off = (j * U + u) * L
                x = buf[pl.ds(off, L)]
                m = x >= t8v
                buf[pl.ds(off, L)] = jnp.where(m, x, 0.0) + kv
                cnt = cnt + m.astype(jnp.int32)
            return cnt

        cnt = lax.fori_loop(0, NV // U, p2, jnp.zeros((L,), jnp.int32))
        count = jnp.sum(cnt)

        @pl.when(count != KTOP)
        def _fallback():
            pltpu.sync_copy(in_hbm.at[row], buf)

            def cond(st):
                return st[1] < KTOP

            def body(st):
                tc, cgt, _tp, _cp = st
                tcv = jnp.full((L,), tc)

                def pw(j, c2):
                    mv, ce = c2
                    for u in range(U):
                        x = buf[pl.ds((j * U + u) * L, L)]
                        mv = jnp.maximum(mv, jnp.where(x < tcv, x, NEG))
                        ce = ce + (x == tcv).astype(jnp.int32)
                    return (mv, ce)

                mv, ce = lax.fori_loop(
                    0, NV // U, pw,
                    (jnp.full((L,), NEG), jnp.zeros((L,), jnp.int32)))
                return (jnp.max(mv), cgt + jnp.sum(ce), tc, cgt)

            tinf = jnp.float32(float("inf"))
            st = lax.while_loop(
                cond, body, (tinf, jnp.int32(0), tinf, jnp.int32(0)))
            t8x = st[2]
            need = KTOP - st[3]
            t8xv = jnp.full((L,), t8x)

            def pr(j, before):
                for u in range(U):
                    off = (j * U + u) * L
                    x = buf[pl.ds(off, L)]
                    gt = x > t8xv
                    eq = x == t8xv
                    eqi = eq.astype(jnp.int32)
                    incl = lax.cumsum(eqi, axis=0)
                    keep = jnp.logical_or(
                        gt, jnp.logical_and(eq, (before + incl) <= need))
                    buf[pl.ds(off, L)] = jnp.where(keep, x, 0.0) + kv
                    before = before + jnp.sum(eqi)
                return before

            lax.fori_loop(0, NV // U, pr, jnp.int32(0))

        pltpu.async_copy(buf, out_hbm.at[row], ssem)

    def pair(i, carry):
        process(base + 2 * i, bufA, lsemA, bufB, lsemB)
        process(base + 2 * i + 1, bufB, lsemB, bufA, lsemA)
        return carry

    lax.fori_loop(0, RPW // 2, pair, jnp.int32(0))

    pltpu.make_async_copy(bufB, out_hbm.at[last], ssem).wait()
    pltpu.make_async_copy(in_hbm.at[last], bufA, lsemA).wait()


def kernel(t, K):
    B, R, Cc = t.shape
    flat = t.reshape(B * R, Cc)
    kvec = jnp.full((L,), jnp.asarray(K, jnp.float32) - float(KTOP),
                    dtype=jnp.float32)
    mesh = plsc.VectorSubcoreMesh(core_axis_name="c", subcore_axis_name="s")
    out = pl.kernel(
        _sc_body,
        out_type=jax.ShapeDtypeStruct((B * R, Cc), jnp.float32),
        mesh=mesh,
        compiler_params=pltpu.CompilerParams(needs_layout_passes=False),
        scratch_types=[
            pltpu.VMEM((C,), jnp.float32),
            pltpu.VMEM((C,), jnp.float32),
            pltpu.VMEM((L,), jnp.float32),
            pltpu.SemaphoreType.DMA,
            pltpu.SemaphoreType.DMA,
            pltpu.SemaphoreType.DMA,
        ],
    )(flat, kvec)
    return out.reshape(B, R, Cc)
